# baseline (speedup 1.0000x reference)
import functools

import jax
import jax.numpy as jnp
from jax import lax
from jax.experimental import pallas as pl
from jax.experimental.pallas import tpu as pltpu
from jax.experimental.pallas import tpu_sc as plsc

B = 16
Q = 900
C = 91
QC = Q * C
QCPAD = 81920
NVEC = QCPAD // 16
NSEL = 300
NPAD = 304
NSELV = NPAD // 16
CANDBUF = 320
CAP16 = 2048
INT_MIN = -(2 ** 31)
INT_MAX = 2 ** 31 - 1


def _sortkey(v):
    u = lax.bitcast_convert_type(v, jnp.int32)
    return u ^ lax.shift_right_logical(lax.shift_right_arithmetic(u, 31), 1)


def _store1(ref, pos, val, lane):
    plsc.store_scatter(ref, [jnp.broadcast_to(pos, (16,))],
                       jnp.broadcast_to(val, (16,)), mask=lane == 0)


def _popcnt(mask):
    return plsc.all_reduce_population_count(mask)[0]


def _body(logits_h, boxes_h, scale_h,
          scores_h, labels_h, boxesout_h, keep_h,
          slab, h0, h1, h2, h3, cand_key, cand_idx,
          gt_key, gt_idx, eq_idx, bx,
          sel_key, sel_idx, x0a, y0a, x1a, y1a, areaa,
          keepv, aosb, scorev, labelv, scalev):
    cid = lax.axis_index("c")
    sid = lax.axis_index("s")
    wid = sid * 2 + cid

    lane = lax.iota(jnp.int32, 16)
    ones_i = jnp.ones((16,), jnp.int32)
    zeros_i = jnp.zeros((16,), jnp.int32)
    hists = (h0, h1, h2, h3)

    @pl.when(wid < B)
    def _work():
        img = wid
        pltpu.sync_copy(logits_h.at[pl.ds(img * QCPAD, QCPAD)], slab)
        pltpu.sync_copy(boxes_h.at[pl.ds(img * (Q * 4), Q * 4)], bx)
        pltpu.sync_copy(scale_h.at[pl.ds(img * 16, 16)], scalev)

        def _zero_hists():
            def _z(k, _):
                for h in hists:
                    h[pl.ds(k * 16, 16)] = zeros_i
                return 0
            lax.fori_loop(0, 256, _z, 0)

        def _slab_hist(sh, prefix, level):
            def _hb(i, _):
                for j, h in enumerate(hists):
                    key = _sortkey(slab[pl.ds(i * 64 + j * 16, 16)])
                    b = lax.shift_right_arithmetic(key, sh) & 0xFF
                    if level == 0:
                        b = b ^ 128
                        mask = lane < 16
                    else:
                        mask = lax.shift_right_arithmetic(key, sh + 8) == prefix
                    addr = b * 16 + lane
                    cnt = plsc.load_gather(h, [addr])
                    plsc.store_scatter(h, [addr], cnt + 1, mask=mask)
                return 0
            lax.fori_loop(0, NVEC // 4, _hb, 0)

        def _cand_hist(sh, prefix, candcnt):
            nv = lax.shift_right_logical(candcnt + 15, 4)

            def _hb(i, _):
                key = cand_key[pl.ds(i * 16, 16)]
                mask = jnp.logical_and(
                    (i * 16 + lane) < candcnt,
                    lax.shift_right_arithmetic(key, sh + 8) == prefix)
                b = lax.shift_right_arithmetic(key, sh) & 0xFF
                addr = b * 16 + lane
                cnt = plsc.load_gather(h0, [addr])
                plsc.store_scatter(h0, [addr], cnt + 1, mask=mask)
                return 0
            lax.fori_loop(0, nv, _hb, 0)

        def _scan_level(n_above):
            def _s(k, carry):
                cum, bsel, above, csel, found = carry
                bb = 255 - k
                dsb = pl.ds(bb * 16, 16)
                row = h0[dsb] + h1[dsb] + h2[dsb] + h3[dsb]
                cnt = jnp.sum(row)
                newcum = cum + cnt
                hit = jnp.logical_and(jnp.logical_not(found),
                                      n_above + newcum >= NSEL)
                bsel = jnp.where(hit, bb, bsel)
                above = jnp.where(hit, cum, above)
                csel = jnp.where(hit, cnt, csel)
                found = jnp.logical_or(found, hit)
                return newcum, bsel, above, csel, found
            _, bsel, above, csel, _ = lax.fori_loop(
                0, 256, _s,
                (jnp.int32(0), jnp.int32(0), jnp.int32(0), jnp.int32(0),
                 False))
            return bsel, above, csel

        with jax.named_scope("ph_hist1"):
            _zero_hists()
            _slab_hist(24, None, 0)
        bsel, above, csel = _scan_level(jnp.int32(0))
        n_above = above
        prefix = bsel - 128

        with jax.named_scope("ph_hist2"):
            _zero_hists()
            _slab_hist(16, prefix, 1)
        bsel, above, csel = _scan_level(n_above)
        n_above = n_above + above
        prefix = lax.shift_left(prefix, 8) | bsel

        candtot = n_above + csel
        collected = candtot <= CAP16

        with jax.named_scope("ph_candcollect"):
            @pl.when(collected)
            def _():
                def _cc(i, cc):
                    key = _sortkey(slab[pl.ds(i * 16, 16)])
                    m = lax.shift_right_arithmetic(key, 16) >= prefix
                    inc = _popcnt(m)

                    @pl.when(inc > 0)
                    def _():
                        plsc.store_compressed(cand_key.at[pl.ds(cc, 16)], key,
                                              mask=m)
                        plsc.store_compressed(cand_idx.at[pl.ds(cc, 16)],
                                              i * 16 + lane, mask=m)
                    return cc + inc
                lax.fori_loop(0, NVEC, _cc, jnp.int32(0))

        for sh, level in ((8, 2), (0, 3)):
            _zero_hists()

            @pl.when(collected)
            def _(sh=sh, prefix=prefix):
                _cand_hist(sh, prefix, candtot)

            @pl.when(jnp.logical_not(collected))
            def _(sh=sh, level=level, prefix=prefix):
                _slab_hist(sh, prefix, level)

            bsel, above, csel = _scan_level(n_above)
            n_above = n_above + above
            prefix = lax.shift_left(prefix, 8) | bsel

        t = prefix
        n_gt = n_above

        def _clr(k, _):
            gt_key[pl.ds(k * 16, 16)] = jnp.full((16,), INT_MIN, jnp.int32)
            return 0
        lax.fori_loop(0, CANDBUF // 16, _clr, 0)

        def _appends(key, idxv, mgt, meq, cgt, ceq):
            inc_gt = _popcnt(mgt)

            @pl.when(inc_gt > 0)
            def _():
                plsc.store_compressed(gt_key.at[pl.ds(cgt, 16)], key,
                                      mask=mgt)
                plsc.store_compressed(gt_idx.at[pl.ds(cgt, 16)], idxv,
                                      mask=mgt)
            inc_eq = _popcnt(meq)

            @pl.when(jnp.logical_and(inc_eq > 0, ceq < NSEL))
            def _():
                plsc.store_compressed(eq_idx.at[pl.ds(ceq, 16)], idxv,
                                      mask=meq)
            return (cgt + inc_gt,
                    jnp.where(ceq < NSEL, ceq + inc_eq, ceq))

        @pl.when(collected)
        def _():
            nv = lax.shift_right_logical(candtot + 15, 4)

            def _fc(i, carry):
                key = cand_key[pl.ds(i * 16, 16)]
                idxv = cand_idx[pl.ds(i * 16, 16)]
                mpos = (i * 16 + lane) < candtot
                mgt = jnp.logical_and(mpos, key > t)
                meq = jnp.logical_and(mpos, key == t)
                return _appends(key, idxv, mgt, meq, *carry)
            lax.fori_loop(0, nv, _fc, (jnp.int32(0), jnp.int32(0)))

        @pl.when(jnp.logical_not(collected))
        def _():
            def _fc(i, carry):
                key = _sortkey(slab[pl.ds(i * 16, 16)])
                idxv = i * 16 + lane
                return _appends(key, idxv, key > t, key == t, *carry)
            lax.fori_loop(0, NVEC, _fc, (jnp.int32(0), jnp.int32(0)))

        sel_idx[pl.ds(288, 16)] = zeros_i
        sel_key[pl.ds(288, 16)] = zeros_i

        def _extract(k, _):
            in_gt = k < n_gt

            @pl.when(in_gt)
            def _():
                mv = gt_key[pl.ds(0, 16)]
                for v in range(1, CANDBUF // 16):
                    mv = jnp.maximum(mv, gt_key[pl.ds(v * 16, 16)])
                m = jnp.max(mv)
                sv = jnp.full((16,), INT_MAX, jnp.int32)
                for v in range(CANDBUF // 16):
                    kv = gt_key[pl.ds(v * 16, 16)]
                    iv = gt_idx[pl.ds(v * 16, 16)]
                    sv = jnp.minimum(sv, jnp.where(kv == m, iv, INT_MAX))
                bidx = jnp.min(sv)
                for v in range(CANDBUF // 16):
                    kv = gt_key[pl.ds(v * 16, 16)]
                    iv = gt_idx[pl.ds(v * 16, 16)]
                    hitv = jnp.logical_and(kv == m, iv == bidx)
                    gt_key[pl.ds(v * 16, 16)] = jnp.where(hitv, INT_MIN, kv)
                _store1(sel_key, k, m, lane)
                _store1(sel_idx, k, bidx, lane)

            @pl.when(jnp.logical_not(in_gt))
            def _():
                _store1(sel_key, k, t, lane)
                _store1(sel_idx, k, eq_idx[pl.ds(k - n_gt, 16)][0], lane)
            return 0
        with jax.named_scope("ph_extract"):
            lax.fori_loop(0, NSEL, _extract, 0)

        srow = scalev[pl.ds(0, 16)]
        W = srow[0]
        H = srow[1]

        def _boxes(v, _):
            dsv = pl.ds(v * 16, 16)
            kv = sel_key[dsv]
            iv = sel_idx[dsv]
            ub = kv ^ lax.shift_right_logical(
                lax.shift_right_arithmetic(kv, 31), 1)
            x = lax.bitcast_convert_type(ub, jnp.float32)
            scorev[dsv] = 1.0 / (1.0 + jnp.exp(-x))
            q0 = (iv.astype(jnp.float32) * jnp.float32(1.0 / C)).astype(jnp.int32)
            r = iv - q0 * C
            q = q0 + (r >= C).astype(jnp.int32) - (r < 0).astype(jnp.int32)
            labelv[dsv] = iv - q * C
            fi = q * 4
            cx = plsc.load_gather(bx, [fi])
            cy = plsc.load_gather(bx, [fi + 1])
            w = plsc.load_gather(bx, [fi + 2])
            h = plsc.load_gather(bx, [fi + 3])
            x0 = (cx - 0.5 * w) * W
            y0 = (cy - 0.5 * h) * H
            x1 = (cx + 0.5 * w) * W
            y1 = (cy + 0.5 * h) * H
            x0a[dsv] = x0
            y0a[dsv] = y0
            x1a[dsv] = x1
            y1a[dsv] = y1
            areaa[dsv] = (x1 - x0) * (y1 - y0)
            keepv[dsv] = ones_i
            pos = (v * 16 + lane) * 4
            plsc.store_scatter(aosb, [pos], x0)
            plsc.store_scatter(aosb, [pos + 1], y0)
            plsc.store_scatter(aosb, [pos + 2], x1)
            plsc.store_scatter(aosb, [pos + 3], y1)
            return 0
        with jax.named_scope("ph_boxes"):
            lax.fori_loop(0, NSELV, _boxes, 0)

        def _nms(i, _):
            @pl.when(keepv[pl.ds(i, 16)][0] != 0)
            def _():
                bx0 = x0a[pl.ds(i, 16)][0]
                by0 = y0a[pl.ds(i, 16)][0]
                bx1 = x1a[pl.ds(i, 16)][0]
                by1 = y1a[pl.ds(i, 16)][0]
                ai = areaa[pl.ds(i, 16)][0]
                for v in range(NSELV):
                    dsv = pl.ds(v * 16, 16)
                    iw = jnp.maximum(
                        jnp.minimum(bx1, x1a[dsv]) - jnp.maximum(bx0, x0a[dsv]),
                        0.0)
                    ih = jnp.maximum(
                        jnp.minimum(by1, y1a[dsv]) - jnp.maximum(by0, y0a[dsv]),
                        0.0)
                    inter = iw * ih
                    union = ai + areaa[dsv] - inter
                    iou = inter / jnp.maximum(union, 1e-9)
                    jv = v * 16 + lane
                    supp = jnp.logical_and(iou > 0.5, jv > i)
                    keepv[dsv] = jnp.where(supp, 0, keepv[dsv])
            return 0
        with jax.named_scope("ph_nms"):
            lax.fori_loop(0, NSEL, _nms, 0)

        pltpu.sync_copy(scorev, scores_h.at[pl.ds(img * NPAD, NPAD)])
        pltpu.sync_copy(labelv, labels_h.at[pl.ds(img * NPAD, NPAD)])
        pltpu.sync_copy(keepv.at[pl.ds(0, NPAD)],
                        keep_h.at[pl.ds(img * NPAD, NPAD)])
        pltpu.sync_copy(aosb, boxesout_h.at[pl.ds(img * NPAD * 4, NPAD * 4)])


_mesh = plsc.VectorSubcoreMesh(core_axis_name="c", subcore_axis_name="s")

_sc_call = functools.partial(
    pl.kernel,
    out_type=(
        jax.ShapeDtypeStruct((B * NPAD,), jnp.float32),
        jax.ShapeDtypeStruct((B * NPAD,), jnp.int32),
        jax.ShapeDtypeStruct((B * NPAD * 4,), jnp.float32),
        jax.ShapeDtypeStruct((B * NPAD,), jnp.int32),
    ),
    mesh=_mesh,
    compiler_params=pltpu.CompilerParams(needs_layout_passes=False),
    scratch_types=(
        pltpu.VMEM((QCPAD,), jnp.float32),
        pltpu.VMEM((4096,), jnp.int32),
        pltpu.VMEM((4096,), jnp.int32),
        pltpu.VMEM((4096,), jnp.int32),
        pltpu.VMEM((4096,), jnp.int32),
        pltpu.VMEM((CAP16 + 16,), jnp.int32),
        pltpu.VMEM((CAP16 + 16,), jnp.int32),
        pltpu.VMEM((CANDBUF,), jnp.int32),
        pltpu.VMEM((CANDBUF,), jnp.int32),
        pltpu.VMEM((CANDBUF,), jnp.int32),
        pltpu.VMEM((Q * 4,), jnp.float32),
        pltpu.VMEM((NPAD,), jnp.int32),
        pltpu.VMEM((NPAD,), jnp.int32),
        pltpu.VMEM((NPAD + 16,), jnp.float32),
        pltpu.VMEM((NPAD + 16,), jnp.float32),
        pltpu.VMEM((NPAD + 16,), jnp.float32),
        pltpu.VMEM((NPAD + 16,), jnp.float32),
        pltpu.VMEM((NPAD + 16,), jnp.float32),
        pltpu.VMEM((NPAD + 16,), jnp.int32),
        pltpu.VMEM((NPAD * 4,), jnp.float32),
        pltpu.VMEM((NPAD,), jnp.float32),
        pltpu.VMEM((NPAD,), jnp.int32),
        pltpu.VMEM((16,), jnp.float32),
    ),
)(_body)


@jax.jit
def kernel(pred_logits, pred_boxes, target_sizes):
    logits = jnp.pad(pred_logits.reshape(B, QC), ((0, 0), (0, QCPAD - QC)),
                     constant_values=-jnp.inf).reshape(-1)
    boxes_in = pred_boxes.reshape(-1)
    ts = target_sizes.astype(jnp.float32)
    scale = jnp.pad(jnp.stack([ts[:, 1], ts[:, 0]], axis=1),
                    ((0, 0), (0, 14))).reshape(-1)
    scores, labels, boxes, keep = _sc_call(logits, boxes_in, scale)
    scores = scores.reshape(B, NPAD)[:, :NSEL]
    labels = labels.reshape(B, NPAD)[:, :NSEL]
    boxes = boxes.reshape(B, NPAD, 4)[:, :NSEL]
    keep = keep.reshape(B, NPAD)[:, :NSEL].astype(bool)
    return scores, labels, boxes, keep


# speedup vs baseline: 1.3404x; 1.3404x over previous
import functools

import jax
import jax.numpy as jnp
from jax import lax
from jax.experimental import pallas as pl
from jax.experimental.pallas import tpu as pltpu
from jax.experimental.pallas import tpu_sc as plsc

B = 16
Q = 900
C = 91
QC = Q * C
QCPAD = 81920
NVEC = QCPAD // 16
NSEL = 300
NPAD = 304
NSELV = NPAD // 16
CANDBUF = 320
CAP8 = 4096
INT_MIN = -(2 ** 31)
INT_MAX = 2 ** 31 - 1


def _sortkey(v):
    u = lax.bitcast_convert_type(v, jnp.int32)
    return u ^ lax.shift_right_logical(lax.shift_right_arithmetic(u, 31), 1)


def _store1(ref, pos, val, lane):
    plsc.store_scatter(ref, [jnp.broadcast_to(pos, (16,))],
                       jnp.broadcast_to(val, (16,)), mask=lane == 0)


def _popcnt(mask):
    return plsc.all_reduce_population_count(mask)[0]


def _body(logits_h, boxes_h, scale_h,
          scores_h, labels_h, boxesout_h, keep_h,
          slab, h0, h1, h2, h3, cand_key, cand_idx,
          gt_key, gt_idx, eq_idx, bx,
          sel_key, sel_idx, x0a, y0a, x1a, y1a, areaa,
          keepv, aosb, scorev, labelv, scalev):
    cid = lax.axis_index("c")
    sid = lax.axis_index("s")
    wid = sid * 2 + cid

    lane = lax.iota(jnp.int32, 16)
    ones_i = jnp.ones((16,), jnp.int32)
    zeros_i = jnp.zeros((16,), jnp.int32)
    hists = (h0, h1, h2, h3)

    @pl.when(wid < B)
    def _work():
        img = wid
        pltpu.sync_copy(logits_h.at[pl.ds(img * QCPAD, QCPAD)], slab)
        pltpu.sync_copy(boxes_h.at[pl.ds(img * (Q * 4), Q * 4)], bx)
        pltpu.sync_copy(scale_h.at[pl.ds(img * 16, 16)], scalev)

        def _zero_hists():
            def _z(k, _):
                for h in hists:
                    h[pl.ds(k * 16, 16)] = zeros_i
                return 0
            lax.fori_loop(0, 256, _z, 0)

        def _slab_hist(sh, prefix, level):
            def _hb(i, _):
                for j, h in enumerate(hists):
                    key = _sortkey(slab[pl.ds(i * 64 + j * 16, 16)])
                    b = lax.shift_right_arithmetic(key, sh) & 0xFF
                    if level == 0:
                        b = b ^ 128
                        mask = lane < 16
                    else:
                        mask = lax.shift_right_arithmetic(key, sh + 8) == prefix
                    addr = b * 16 + lane
                    cnt = plsc.load_gather(h, [addr])
                    plsc.store_scatter(h, [addr], cnt + 1, mask=mask)
                return 0
            lax.fori_loop(0, NVEC // 4, _hb, 0)

        def _cand_hist(sh, prefix, candcnt):
            nv = lax.shift_right_logical(candcnt + 15, 4)

            def _hb(i, _):
                key = cand_key[pl.ds(i * 16, 16)]
                mask = jnp.logical_and(
                    (i * 16 + lane) < candcnt,
                    lax.shift_right_arithmetic(key, sh + 8) == prefix)
                b = lax.shift_right_arithmetic(key, sh) & 0xFF
                addr = b * 16 + lane
                cnt = plsc.load_gather(h0, [addr])
                plsc.store_scatter(h0, [addr], cnt + 1, mask=mask)
                return 0
            lax.fori_loop(0, nv, _hb, 0)

        def _scan_level(n_above):
            def _s(k, carry):
                cum, bsel, above, csel, found = carry
                bb = 255 - k
                dsb = pl.ds(bb * 16, 16)
                row = h0[dsb] + h1[dsb] + h2[dsb] + h3[dsb]
                cnt = jnp.sum(row)
                newcum = cum + cnt
                hit = jnp.logical_and(jnp.logical_not(found),
                                      n_above + newcum >= NSEL)
                bsel = jnp.where(hit, bb, bsel)
                above = jnp.where(hit, cum, above)
                csel = jnp.where(hit, cnt, csel)
                found = jnp.logical_or(found, hit)
                return newcum, bsel, above, csel, found
            _, bsel, above, csel, _ = lax.fori_loop(
                0, 256, _s,
                (jnp.int32(0), jnp.int32(0), jnp.int32(0), jnp.int32(0),
                 False))
            return bsel, above, csel

        _zero_hists()
        _slab_hist(24, None, 0)
        bsel, above, csel = _scan_level(jnp.int32(0))
        n_above = above
        prefix = bsel - 128

        candtot = n_above + csel
        collected = candtot <= CAP8

        @pl.when(collected)
        def _():
            def _cc(i, cc):
                ks = [_sortkey(slab[pl.ds(i * 64 + j * 16, 16)])
                      for j in range(4)]
                ms = [lax.shift_right_arithmetic(k, 24) >= prefix for k in ks]
                incs = [_popcnt(m) for m in ms]
                tot = incs[0] + incs[1] + incs[2] + incs[3]

                @pl.when(tot > 0)
                def _():
                    off = cc
                    for j in range(4):
                        plsc.store_compressed(cand_key.at[pl.ds(off, 16)],
                                              ks[j], mask=ms[j])
                        plsc.store_compressed(cand_idx.at[pl.ds(off, 16)],
                                              i * 64 + j * 16 + lane,
                                              mask=ms[j])
                        off = off + incs[j]
                return cc + tot
            lax.fori_loop(0, NVEC // 4, _cc, jnp.int32(0))

        for sh, level in ((16, 1), (8, 2), (0, 3)):
            _zero_hists()

            @pl.when(collected)
            def _(sh=sh, prefix=prefix):
                _cand_hist(sh, prefix, candtot)

            @pl.when(jnp.logical_not(collected))
            def _(sh=sh, level=level, prefix=prefix):
                _slab_hist(sh, prefix, level)

            bsel, above, csel = _scan_level(n_above)
            n_above = n_above + above
            prefix = lax.shift_left(prefix, 8) | bsel

        t = prefix
        n_gt = n_above

        def _clr(k, _):
            gt_key[pl.ds(k * 16, 16)] = jnp.full((16,), INT_MIN, jnp.int32)
            return 0
        lax.fori_loop(0, CANDBUF // 16, _clr, 0)

        def _appends(key, idxv, mgt, meq, cgt, ceq):
            inc_gt = _popcnt(mgt)

            @pl.when(inc_gt > 0)
            def _():
                plsc.store_compressed(gt_key.at[pl.ds(cgt, 16)], key,
                                      mask=mgt)
                plsc.store_compressed(gt_idx.at[pl.ds(cgt, 16)], idxv,
                                      mask=mgt)
            inc_eq = _popcnt(meq)

            @pl.when(jnp.logical_and(inc_eq > 0, ceq < NSEL))
            def _():
                plsc.store_compressed(eq_idx.at[pl.ds(ceq, 16)], idxv,
                                      mask=meq)
            return (cgt + inc_gt,
                    jnp.where(ceq < NSEL, ceq + inc_eq, ceq))

        @pl.when(collected)
        def _():
            nv = lax.shift_right_logical(candtot + 15, 4)

            def _fc(i, carry):
                key = cand_key[pl.ds(i * 16, 16)]
                idxv = cand_idx[pl.ds(i * 16, 16)]
                mpos = (i * 16 + lane) < candtot
                mgt = jnp.logical_and(mpos, key > t)
                meq = jnp.logical_and(mpos, key == t)
                return _appends(key, idxv, mgt, meq, *carry)
            lax.fori_loop(0, nv, _fc, (jnp.int32(0), jnp.int32(0)))

        @pl.when(jnp.logical_not(collected))
        def _():
            def _fc(i, carry):
                key = _sortkey(slab[pl.ds(i * 16, 16)])
                idxv = i * 16 + lane
                return _appends(key, idxv, key > t, key == t, *carry)
            lax.fori_loop(0, NVEC, _fc, (jnp.int32(0), jnp.int32(0)))

        sel_idx[pl.ds(288, 16)] = zeros_i
        sel_key[pl.ds(288, 16)] = zeros_i

        def _extract(k, _):
            in_gt = k < n_gt

            @pl.when(in_gt)
            def _():
                mv = gt_key[pl.ds(0, 16)]
                for v in range(1, CANDBUF // 16):
                    mv = jnp.maximum(mv, gt_key[pl.ds(v * 16, 16)])
                m = jnp.max(mv)
                sv = jnp.full((16,), INT_MAX, jnp.int32)
                for v in range(CANDBUF // 16):
                    kv = gt_key[pl.ds(v * 16, 16)]
                    iv = gt_idx[pl.ds(v * 16, 16)]
                    sv = jnp.minimum(sv, jnp.where(kv == m, iv, INT_MAX))
                bidx = jnp.min(sv)
                for v in range(CANDBUF // 16):
                    kv = gt_key[pl.ds(v * 16, 16)]
                    iv = gt_idx[pl.ds(v * 16, 16)]
                    hitv = jnp.logical_and(kv == m, iv == bidx)
                    gt_key[pl.ds(v * 16, 16)] = jnp.where(hitv, INT_MIN, kv)
                _store1(sel_key, k, m, lane)
                _store1(sel_idx, k, bidx, lane)

            @pl.when(jnp.logical_not(in_gt))
            def _():
                _store1(sel_key, k, t, lane)
                _store1(sel_idx, k, eq_idx[pl.ds(k - n_gt, 16)][0], lane)
            return 0
        lax.fori_loop(0, NSEL, _extract, 0)

        srow = scalev[pl.ds(0, 16)]
        W = srow[0]
        H = srow[1]

        def _boxes(v, _):
            dsv = pl.ds(v * 16, 16)
            kv = sel_key[dsv]
            iv = sel_idx[dsv]
            ub = kv ^ lax.shift_right_logical(
                lax.shift_right_arithmetic(kv, 31), 1)
            x = lax.bitcast_convert_type(ub, jnp.float32)
            scorev[dsv] = 1.0 / (1.0 + jnp.exp(-x))
            q0 = (iv.astype(jnp.float32) * jnp.float32(1.0 / C)).astype(jnp.int32)
            r = iv - q0 * C
            q = q0 + (r >= C).astype(jnp.int32) - (r < 0).astype(jnp.int32)
            labelv[dsv] = iv - q * C
            fi = q * 4
            cx = plsc.load_gather(bx, [fi])
            cy = plsc.load_gather(bx, [fi + 1])
            w = plsc.load_gather(bx, [fi + 2])
            h = plsc.load_gather(bx, [fi + 3])
            x0 = (cx - 0.5 * w) * W
            y0 = (cy - 0.5 * h) * H
            x1 = (cx + 0.5 * w) * W
            y1 = (cy + 0.5 * h) * H
            x0a[dsv] = x0
            y0a[dsv] = y0
            x1a[dsv] = x1
            y1a[dsv] = y1
            areaa[dsv] = (x1 - x0) * (y1 - y0)
            keepv[dsv] = ones_i
            pos = (v * 16 + lane) * 4
            plsc.store_scatter(aosb, [pos], x0)
            plsc.store_scatter(aosb, [pos + 1], y0)
            plsc.store_scatter(aosb, [pos + 2], x1)
            plsc.store_scatter(aosb, [pos + 3], y1)
            return 0
        lax.fori_loop(0, NSELV, _boxes, 0)

        def _nms(i, _):
            @pl.when(keepv[pl.ds(i, 16)][0] != 0)
            def _():
                bx0 = x0a[pl.ds(i, 16)][0]
                by0 = y0a[pl.ds(i, 16)][0]
                bx1 = x1a[pl.ds(i, 16)][0]
                by1 = y1a[pl.ds(i, 16)][0]
                ai = areaa[pl.ds(i, 16)][0]

                def _inner(v, _):
                    dsv = pl.ds(v * 16, 16)
                    iw = jnp.maximum(
                        jnp.minimum(bx1, x1a[dsv]) - jnp.maximum(bx0, x0a[dsv]),
                        0.0)
                    ih = jnp.maximum(
                        jnp.minimum(by1, y1a[dsv]) - jnp.maximum(by0, y0a[dsv]),
                        0.0)
                    inter = iw * ih
                    union = ai + areaa[dsv] - inter
                    iou = inter / jnp.maximum(union, 1e-9)
                    jv = v * 16 + lane
                    supp = jnp.logical_and(iou > 0.5, jv > i)
                    keepv[dsv] = jnp.where(supp, 0, keepv[dsv])
                    return 0
                lax.fori_loop(lax.shift_right_logical(i, 4), NSELV, _inner, 0)
            return 0
        lax.fori_loop(0, NSEL, _nms, 0)

        pltpu.sync_copy(scorev, scores_h.at[pl.ds(img * NPAD, NPAD)])
        pltpu.sync_copy(labelv, labels_h.at[pl.ds(img * NPAD, NPAD)])
        pltpu.sync_copy(keepv.at[pl.ds(0, NPAD)],
                        keep_h.at[pl.ds(img * NPAD, NPAD)])
        pltpu.sync_copy(aosb, boxesout_h.at[pl.ds(img * NPAD * 4, NPAD * 4)])


_mesh = plsc.VectorSubcoreMesh(core_axis_name="c", subcore_axis_name="s")

_sc_call = functools.partial(
    pl.kernel,
    out_type=(
        jax.ShapeDtypeStruct((B * NPAD,), jnp.float32),
        jax.ShapeDtypeStruct((B * NPAD,), jnp.int32),
        jax.ShapeDtypeStruct((B * NPAD * 4,), jnp.float32),
        jax.ShapeDtypeStruct((B * NPAD,), jnp.int32),
    ),
    mesh=_mesh,
    compiler_params=pltpu.CompilerParams(needs_layout_passes=False),
    scratch_types=(
        pltpu.VMEM((QCPAD,), jnp.float32),
        pltpu.VMEM((4096,), jnp.int32),
        pltpu.VMEM((4096,), jnp.int32),
        pltpu.VMEM((4096,), jnp.int32),
        pltpu.VMEM((4096,), jnp.int32),
        pltpu.VMEM((CAP8 + 16,), jnp.int32),
        pltpu.VMEM((CAP8 + 16,), jnp.int32),
        pltpu.VMEM((CANDBUF,), jnp.int32),
        pltpu.VMEM((CANDBUF,), jnp.int32),
        pltpu.VMEM((CANDBUF,), jnp.int32),
        pltpu.VMEM((Q * 4,), jnp.float32),
        pltpu.VMEM((NPAD,), jnp.int32),
        pltpu.VMEM((NPAD,), jnp.int32),
        pltpu.VMEM((NPAD + 16,), jnp.float32),
        pltpu.VMEM((NPAD + 16,), jnp.float32),
        pltpu.VMEM((NPAD + 16,), jnp.float32),
        pltpu.VMEM((NPAD + 16,), jnp.float32),
        pltpu.VMEM((NPAD + 16,), jnp.float32),
        pltpu.VMEM((NPAD + 16,), jnp.int32),
        pltpu.VMEM((NPAD * 4,), jnp.float32),
        pltpu.VMEM((NPAD,), jnp.float32),
        pltpu.VMEM((NPAD,), jnp.int32),
        pltpu.VMEM((16,), jnp.float32),
    ),
)(_body)


@jax.jit
def kernel(pred_logits, pred_boxes, target_sizes):
    logits = jnp.pad(pred_logits.reshape(B, QC), ((0, 0), (0, QCPAD - QC)),
                     constant_values=-jnp.inf).reshape(-1)
    boxes_in = pred_boxes.reshape(-1)
    ts = target_sizes.astype(jnp.float32)
    scale = jnp.pad(jnp.stack([ts[:, 1], ts[:, 0]], axis=1),
                    ((0, 0), (0, 14))).reshape(-1)
    scores, labels, boxes, keep = _sc_call(logits, boxes_in, scale)
    scores = scores.reshape(B, NPAD)[:, :NSEL]
    labels = labels.reshape(B, NPAD)[:, :NSEL]
    boxes = boxes.reshape(B, NPAD, 4)[:, :NSEL]
    keep = keep.reshape(B, NPAD)[:, :NSEL].astype(bool)
    return scores, labels, boxes, keep
